# Initial kernel scaffold; baseline (speedup 1.0000x reference)
import jax
import jax.numpy as jnp
from jax.experimental import pallas as pl


def kernel(batch, mf_w1, mf_b1, mf_w2, mf_b2, c_w1, c_b1, c_w2, c_b2):
    raise NotImplementedError("write your pallas kernel here")


# speedup vs baseline: 15.4194x; 15.4194x over previous
import functools

import jax
import jax.numpy as jnp
from jax import lax
from jax.experimental import pallas as pl
from jax.experimental.pallas import tpu as pltpu

MAX_DISPLACEMENT = 4.0
RADIUS = 4


def _shift(x, off):
    if off == 0:
        return x
    size = x.shape[-1]
    return pltpu.roll(x, shift=(-off) % size, axis=len(x.shape) - 1)


def _edge_masks(H, W, T):
    idx = lax.broadcasted_iota(jnp.int32, (1, T), 1)
    row = idx // W
    col = idx - row * W
    rowm = []
    colm = []
    for d in (-1, 0, 1):
        rowm.append(((row + d >= 0) & (row + d <= H - 1)).astype(jnp.float32))
        colm.append(((col + d >= 0) & (col + d <= W - 1)).astype(jnp.float32))
    return rowm, colm


def _fused_conv(x, w1dx_ref, b1_ref, w2m_ref, b2_ref, rowm, colm, H, W):
    cin, T = x.shape
    hid = w1dx_ref.shape[1]
    nout = w2m_ref.shape[0] // 9

    parts = []
    for dy in range(3):
        p = _shift(x, (dy - 1) * W)
        if dy != 1:
            p = p * rowm[dy]
        parts.append(p)
    stacked = jnp.concatenate(parts, axis=0)

    acc = jnp.zeros((hid, T), jnp.float32)
    for dx in range(3):
        opnd = _shift(stacked, dx - 1)
        if dx != 1:
            opnd = opnd * colm[dx]
        acc = acc + jnp.dot(w1dx_ref[dx], opnd,
                            preferred_element_type=jnp.float32)
    h = jnp.maximum(acc + b1_ref[...], 0.0)

    z = jnp.dot(w2m_ref[...], h, preferred_element_type=jnp.float32)
    out = jnp.zeros((nout, T), jnp.float32)
    for dy in range(3):
        for dx in range(3):
            tap = dy * 3 + dx
            zt = _shift(z[tap * nout:(tap + 1) * nout, :],
                        (dy - 1) * W + (dx - 1))
            if dy != 1:
                zt = zt * rowm[dy]
            if dx != 1:
                zt = zt * colm[dx]
            out = out + zt
    return out + b2_ref[...]


def _maskflow_kernel(x_ref, w1_ref, b1_ref, w2_ref, b2_ref, fm_ref, *, H, W):
    T = H * W
    x = x_ref[0]
    rowm, colm = _edge_masks(H, W, T)
    y = _fused_conv(x, w1_ref, b1_ref, w2_ref, b2_ref, rowm, colm, H, W)
    fm_ref[0] = jnp.concatenate(
        [jnp.tanh(y[0:2, :]) * MAX_DISPLACEMENT, jax.nn.sigmoid(y[2:3, :])],
        axis=0)


def _warp_completion_kernel(fm_ref, i2_ref, w1_ref, b1_ref, w2_ref, b2_ref,
                            oh_ref, iw_ref, ic_ref, *, H, W):
    T = H * W
    fm = fm_ref[0]
    img = i2_ref[0]
    C = img.shape[0]

    u = fm[0:1, :]
    v = fm[1:2, :]
    osm = fm[2:3, :]

    idx = lax.broadcasted_iota(jnp.int32, (1, T), 1)
    row = idx // W
    col = idx - row * W

    sx = col.astype(jnp.float32) + u
    x0f = jnp.floor(sx)
    wx = sx - x0f
    x0 = x0f.astype(jnp.int32)
    x0off = x0 - col
    ax0 = (1.0 - wx) * ((x0 >= 0) & (x0 <= W - 1)).astype(jnp.float32)
    ax1 = wx * ((x0 >= -1) & (x0 <= W - 2)).astype(jnp.float32)

    sy = row.astype(jnp.float32) + v
    y0f = jnp.floor(sy)
    wy = sy - y0f
    y0 = y0f.astype(jnp.int32)
    y0off = y0 - row
    ay0 = (1.0 - wy) * ((y0 >= 0) & (y0 <= H - 1)).astype(jnp.float32)
    ay1 = wy * ((y0 >= -1) & (y0 <= H - 2)).astype(jnp.float32)

    wyr = []
    for r in range(-RADIUS, RADIUS + 1):
        wyr.append(jnp.where(y0off == r, ay0, 0.0) +
                   jnp.where(y0off == r - 1, ay1, 0.0))

    iw = jnp.zeros((C, T), jnp.float32)
    for c in range(-RADIUS, RADIUS + 1):
        imgc = _shift(img, c)
        wxc = (jnp.where(x0off == c, ax0, 0.0) +
               jnp.where(x0off == c - 1, ax1, 0.0))
        for r in range(-RADIUS, RADIUS + 1):
            w = wxc * wyr[r + RADIUS]
            iw = iw + w * _shift(imgc, r * W)

    ohard = (osm > 0.5).astype(jnp.float32)
    io = iw * ohard

    rowm, colm = _edge_masks(H, W, T)
    ic = _fused_conv(io, w1_ref, b1_ref, w2_ref, b2_ref, rowm, colm, H, W)

    oh_ref[0] = ohard
    iw_ref[0] = iw
    ic_ref[0] = ic


def _prep_conv_weights(w1, b1, w2, b2):
    taps, cin, hid = w1.shape
    nout = w2.shape[2]
    w1g = w1.reshape(3, 3, cin, hid)
    w1dx = jnp.transpose(w1g, (1, 3, 0, 2)).reshape(3, hid, 3 * cin)
    w2m = jnp.transpose(w2, (0, 2, 1)).reshape(9 * nout, hid)
    return w1dx, b1.reshape(hid, 1), w2m, b2.reshape(nout, 1)


def kernel(batch, mf_w1, mf_b1, mf_w2, mf_b2, c_w1, c_b1, c_w2, c_b2):
    B, two, C, H, W = batch.shape
    T = H * W
    hid = mf_w1.shape[2]

    xb = batch.reshape(B, 2 * C, T)

    mw1, mb1, mw2, mb2 = _prep_conv_weights(mf_w1, mf_b1, mf_w2, mf_b2)
    cw1, cb1, cw2, cb2 = _prep_conv_weights(c_w1, c_b1, c_w2, c_b2)

    cparams = pltpu.CompilerParams(
        dimension_semantics=("parallel",),
        vmem_limit_bytes=64 * 1024 * 1024,
    )

    fm = pl.pallas_call(
        functools.partial(_maskflow_kernel, H=H, W=W),
        out_shape=jax.ShapeDtypeStruct((B, 3, T), jnp.float32),
        grid=(B,),
        in_specs=[
            pl.BlockSpec((1, 2 * C, T), lambda b: (b, 0, 0)),
            pl.BlockSpec(mw1.shape, lambda b: (0, 0, 0)),
            pl.BlockSpec(mb1.shape, lambda b: (0, 0)),
            pl.BlockSpec(mw2.shape, lambda b: (0, 0)),
            pl.BlockSpec(mb2.shape, lambda b: (0, 0)),
        ],
        out_specs=pl.BlockSpec((1, 3, T), lambda b: (b, 0, 0)),
        compiler_params=cparams,
    )(xb, mw1, mb1, mw2, mb2)

    oh, iw, ic = pl.pallas_call(
        functools.partial(_warp_completion_kernel, H=H, W=W),
        out_shape=(jax.ShapeDtypeStruct((B, 1, T), jnp.float32),
                   jax.ShapeDtypeStruct((B, C, T), jnp.float32),
                   jax.ShapeDtypeStruct((B, C, T), jnp.float32)),
        grid=(B,),
        in_specs=[
            pl.BlockSpec((1, 3, T), lambda b: (b, 0, 0)),
            pl.BlockSpec((1, C, T), lambda b: (b, 1, 0)),
            pl.BlockSpec(cw1.shape, lambda b: (0, 0, 0)),
            pl.BlockSpec(cb1.shape, lambda b: (0, 0)),
            pl.BlockSpec(cw2.shape, lambda b: (0, 0)),
            pl.BlockSpec(cb2.shape, lambda b: (0, 0)),
        ],
        out_specs=(pl.BlockSpec((1, 1, T), lambda b: (b, 0, 0)),
                   pl.BlockSpec((1, C, T), lambda b: (b, 0, 0)),
                   pl.BlockSpec((1, C, T), lambda b: (b, 0, 0))),
        compiler_params=cparams,
    )(fm, xb, cw1, cb1, cw2, cb2)

    O_s = fm[:, 2:3, :].reshape(B, 1, H, W)
    O_h = oh.reshape(B, 1, H, W)
    Iw1 = iw.reshape(B, C, H, W)
    Ic1 = ic.reshape(B, C, H, W)
    return O_s, O_h, Ic1, Iw1
